# Initial kernel scaffold; baseline (speedup 1.0000x reference)
import jax
import jax.numpy as jnp
from jax.experimental import pallas as pl


def kernel(X, E, snorm_n, snorm_e, edge_index, params):
    raise NotImplementedError("write your pallas kernel here")


# speedup vs baseline: 1.0099x; 1.0099x over previous
import functools
import jax
import jax.numpy as jnp
from jax.experimental import pallas as pl
from jax.experimental.pallas import tpu as pltpu

N_NODES = 10000
N_EDGES = 320000
D = 128


def _mm_body(x_ref, w_ref, b_ref, o_ref):
    o_ref[...] = (
        jnp.dot(x_ref[...], w_ref[...], preferred_element_type=jnp.float32)
        + b_ref[...]
    )


def _mm(x, wb, block=512):
    W, b = wb
    n = x.shape[0]
    grid = (pl.cdiv(n, block),)
    return pl.pallas_call(
        _mm_body,
        grid=grid,
        in_specs=[
            pl.BlockSpec((block, x.shape[1]), lambda i: (i, 0)),
            pl.BlockSpec((x.shape[1], W.shape[1]), lambda i: (0, 0)),
            pl.BlockSpec((1, W.shape[1]), lambda i: (0, 0)),
        ],
        out_specs=pl.BlockSpec((block, W.shape[1]), lambda i: (i, 0)),
        out_shape=jax.ShapeDtypeStruct((n, W.shape[1]), jnp.float32),
    )(x, W, b.reshape(1, -1))


def _bn(x, gb):
    m = jnp.mean(x, axis=0)
    v = jnp.var(x, axis=0)
    return gb[0] * (x - m) / jnp.sqrt(v + 1e-5) + gb[1]


def kernel(X, E, snorm_n, snorm_e, edge_index, params):
    src = edge_index[0]
    dst = edge_index[1]
    H = _mm(X, params["emb_h"])
    we, be = params["emb_e"]
    Ef = E * we.reshape(1, D) + be.reshape(1, D)
    for lp in params["layers"]:
        AX = _mm(H, lp["A"])
        BX = _mm(H, lp["B"])
        DX = _mm(H, lp["D"])
        EX = _mm(H, lp["E"])
        CE = _mm(Ef, lp["C"])
        e = CE + DX[src] + EX[dst]
        sig = jax.nn.sigmoid(e)
        num = jax.ops.segment_sum(sig * BX[src], dst, num_segments=N_NODES)
        den = jax.ops.segment_sum(sig, dst, num_segments=N_NODES)
        Hn = AX + num / (den + 1e-9)
        Hn = Hn * snorm_n
        En = e * snorm_e
        Hn = jax.nn.relu(_bn(Hn, lp["bnh"]))
        En = jax.nn.relu(_bn(En, lp["bne"]))
        H = H + Hn
        Ef = Ef + En
    y = jnp.mean(H, axis=0, keepdims=True)
    for wb in params["mlp"][:-1]:
        y = jax.nn.relu(_mm(y, wb, block=8))
    y = _mm(y, params["mlp"][-1], block=8)
    return y
